# baseline (speedup 1.0000x reference)
import functools

import jax
import jax.numpy as jnp
from jax import lax
from jax.experimental import pallas as pl
from jax.experimental.pallas import tpu as pltpu
from jax.experimental.pallas import tpu_sc as plsc

NUM_PROMPTS = 1000
SEQ_LEN = 77
TEXT_DIM = 4096
BATCH = 256

R = SEQ_LEN
DC = TEXT_DIM
TOT = BATCH * R
NW = 32
PER = TOT // NW
GRP = 8
STEPS = PER // GRP
PAD = ((PER + 15) // 16) * 16

_mesh = plsc.VectorSubcoreMesh(core_axis_name="c", subcore_axis_name="s")


@functools.partial(
    pl.kernel,
    mesh=_mesh,
    out_type=jax.ShapeDtypeStruct((TOT, DC), jnp.float32),
    compiler_params=pltpu.CompilerParams(needs_layout_passes=False),
    scratch_types=[
        pltpu.VMEM((BATCH,), jnp.int32),
        pltpu.VMEM((PAD,), jnp.int32),
        pltpu.VMEM((GRP, DC), jnp.float32),
        pltpu.SemaphoreType.DMA,
    ],
)
def _sc_gather(table, idx_hbm, out, idx_v, srcids, buf, sem):
    wid = lax.axis_index("s") * 2 + lax.axis_index("c")
    gbase = wid * PER

    pltpu.sync_copy(idx_hbm, idx_v)

    for k in range(PAD // 16):
        g = gbase + k * 16 + lax.iota(jnp.int32, 16)
        g = jnp.minimum(g, TOT - 1)
        b = lax.shift_right_logical(g * 13618, 20)
        c = g - b * R
        rows = plsc.load_gather(idx_v, [b])
        srcids[pl.ds(k * 16, 16)] = rows * R + c

    def step(t, carry):
        pltpu.async_copy(table.at[srcids.at[pl.ds(t * GRP, GRP)]], buf, sem).wait()
        pltpu.sync_copy(buf, out.at[pl.ds(gbase + t * GRP, GRP)])
        return carry

    lax.fori_loop(0, STEPS, step, 0)


def kernel(prompt_idx, embeddings):
    table = embeddings.reshape(NUM_PROMPTS * R, DC)
    out = _sc_gather(table, prompt_idx.astype(jnp.int32))
    return out.reshape(BATCH, SEQ_LEN, TEXT_DIM)


# speedup vs baseline: 5.0614x; 5.0614x over previous
import functools

import jax
import jax.numpy as jnp
from jax import lax
from jax.experimental import pallas as pl
from jax.experimental.pallas import tpu as pltpu
from jax.experimental.pallas import tpu_sc as plsc

NUM_PROMPTS = 1000
SEQ_LEN = 77
TEXT_DIM = 4096
BATCH = 256

NW = 32
ROWS_PER_TILE = BATCH // NW
LB = 512
NBLK = TEXT_DIM // LB

_mesh = plsc.VectorSubcoreMesh(core_axis_name="c", subcore_axis_name="s")


@functools.partial(
    pl.kernel,
    mesh=_mesh,
    out_type=jax.ShapeDtypeStruct((BATCH, SEQ_LEN, TEXT_DIM), jnp.float32),
    compiler_params=pltpu.CompilerParams(needs_layout_passes=False),
    scratch_types=[
        pltpu.VMEM((BATCH + 16,), jnp.int32),
        pltpu.VMEM((1, SEQ_LEN, LB), jnp.float32),
        pltpu.VMEM((1, SEQ_LEN, LB), jnp.float32),
        pltpu.SemaphoreType.DMA,
        pltpu.SemaphoreType.DMA,
        pltpu.SemaphoreType.DMA,
    ],
)
def _sc_gather(table, idx_hbm, out, idx_v, b0, b1, gs, s0, s1):
    wid = lax.axis_index("s") * 2 + lax.axis_index("c")
    rbase = wid * ROWS_PER_TILE

    pltpu.sync_copy(idx_hbm, idx_v.at[pl.ds(0, BATCH)])
    ids = idx_v[pl.ds(rbase, 16)]

    bufs = [(b0, s0), (b1, s1)]
    for r in range(ROWS_PER_TILE):
        row = ids[r]
        for c in range(NBLK):
            u = r * NBLK + c
            buf, sem = bufs[u % 2]
            dst = out.at[pl.ds(rbase + r, 1), :, pl.ds(c * LB, LB)]
            if u >= 2:
                pltpu.make_async_copy(buf, dst, sem).wait()
            pltpu.async_copy(
                table.at[pl.ds(row, 1), :, pl.ds(c * LB, LB)], buf, gs
            ).wait()
            pltpu.async_copy(buf, dst, sem)

    last = ROWS_PER_TILE - 1
    pltpu.make_async_copy(
        b0, out.at[pl.ds(rbase + last, 1), :, pl.ds((NBLK - 2) * LB, LB)],
        s0).wait()
    pltpu.make_async_copy(
        b1, out.at[pl.ds(rbase + last, 1), :, pl.ds((NBLK - 1) * LB, LB)],
        s1).wait()


def kernel(prompt_idx, embeddings):
    return _sc_gather(embeddings, prompt_idx.astype(jnp.int32))


# speedup vs baseline: 5.1017x; 1.0080x over previous
import functools

import jax
import jax.numpy as jnp
from jax import lax
from jax.experimental import pallas as pl
from jax.experimental.pallas import tpu as pltpu
from jax.experimental.pallas import tpu_sc as plsc

NUM_PROMPTS = 1000
SEQ_LEN = 77
TEXT_DIM = 4096
BATCH = 256

NW = 32
ROWS_PER_TILE = BATCH // NW
NBLK = 10
NBUF = 3
UNITS = [(r, c) for r in range(ROWS_PER_TILE) for c in range(NBLK)]

_mesh = plsc.VectorSubcoreMesh(core_axis_name="c", subcore_axis_name="s")


@functools.partial(
    pl.kernel,
    mesh=_mesh,
    out_type=jax.ShapeDtypeStruct((BATCH, SEQ_LEN, TEXT_DIM), jnp.float32),
    compiler_params=pltpu.CompilerParams(needs_layout_passes=False),
    scratch_types=[
        pltpu.VMEM((BATCH + 16,), jnp.int32),
        pltpu.VMEM((1, 8, TEXT_DIM), jnp.float32),
        pltpu.VMEM((1, 8, TEXT_DIM), jnp.float32),
        pltpu.VMEM((1, 8, TEXT_DIM), jnp.float32),
        pltpu.SemaphoreType.DMA,
        pltpu.SemaphoreType.DMA,
        pltpu.SemaphoreType.DMA,
        pltpu.SemaphoreType.DMA,
    ],
)
def _sc_gather(table, idx_hbm, out, idx_v, b0, b1, b2, gs, s0, s1, s2):
    wid = lax.axis_index("s") * 2 + lax.axis_index("c")
    rbase = wid * ROWS_PER_TILE

    pltpu.sync_copy(idx_hbm, idx_v.at[pl.ds(0, BATCH)])
    ids = idx_v[pl.ds(rbase, 16)]

    bufs = [(b0, s0), (b1, s1), (b2, s2)]

    def blk(r, c):
        sl = 8 if c < NBLK - 1 else SEQ_LEN - 8 * (NBLK - 1)
        return r, c * 8, sl

    def src(i):
        r, off, sl = blk(*UNITS[i])
        return table.at[pl.ds(ids[r], 1), pl.ds(off, sl), :]

    def buf_view(i):
        _, _, sl = blk(*UNITS[i])
        buf, sem = bufs[i % NBUF]
        return (buf if sl == 8 else buf.at[:, pl.ds(0, sl), :]), sem

    def dst(i):
        r, off, sl = blk(*UNITS[i])
        return out.at[pl.ds(rbase + r, 1), pl.ds(off, sl), :]

    def start_gather(i):
        bv, _ = buf_view(i)
        if i >= NBUF:
            pv, sem = buf_view(i - NBUF)
            pltpu.make_async_copy(pv, dst(i - NBUF), sem).wait()
        pltpu.async_copy(src(i), bv, gs)

    n = len(UNITS)
    start_gather(0)
    for i in range(n):
        if i + 1 < n:
            start_gather(i + 1)
        bv, sem = buf_view(i)
        pltpu.make_async_copy(src(i), bv, gs).wait()
        pltpu.async_copy(bv, dst(i), sem)

    for i in range(n - NBUF, n):
        pv, sem = buf_view(i)
        pltpu.make_async_copy(pv, dst(i), sem).wait()


def kernel(prompt_idx, embeddings):
    return _sc_gather(embeddings, prompt_idx.astype(jnp.int32))
